# Initial kernel scaffold; baseline (speedup 1.0000x reference)
import jax
import jax.numpy as jnp
from jax.experimental import pallas as pl


def kernel(x, weight, bias, conv_offset_mask_weight, conv_offset_mask_bias):
    raise NotImplementedError("write your pallas kernel here")


# speedup vs baseline: 5.1491x; 5.1491x over previous
import jax
import jax.numpy as jnp
from jax import lax
from jax.experimental import pallas as pl
from jax.experimental.pallas import tpu as pltpu


def _pick_tile(hw):
    for cand in (512, 256, 128):
        if hw % cand == 0 and hw > cand:
            return cand
    return hw


def _make_body(H, W, C, O, tile):
    HW = H * W
    n_taps = 9

    def _body(x_ref, w_all_ref, b_om_ref, w_dcn_ref, b_dcn_ref, base_ref,
              out_ref, idx_ref, mask_ref):
        f32 = jnp.float32
        bf16 = jnp.bfloat16
        x = x_ref[0]

        y_all = jnp.dot(w_all_ref[...], x, preferred_element_type=f32)
        pos = lax.broadcasted_iota(jnp.int32, (1, HW), 1)
        r = pos // W
        c = pos - r * W
        om = jnp.broadcast_to(b_om_ref[...], (32, HW))
        for k in range(n_taps):
            dy, dx = k // 3 - 1, k % 3 - 1
            s = dy * W + dx
            yk = y_all[k * 32:(k + 1) * 32, :]
            rolled = pltpu.roll(yk, (-s) % HW, axis=1) if s else yk
            valid = ((r + dy >= 0) & (r + dy < H)
                     & (c + dx >= 0) & (c + dx < W))
            om = om + jnp.where(valid, rolled, 0.0)

        offset = om[:2 * n_taps]
        maskv = jax.nn.sigmoid(om[2 * n_taps:3 * n_taps])
        idx_ref[0] = base_ref[...] + offset
        mask_ref[0] = maskv

        xw = jnp.dot(w_dcn_ref[...], x.astype(bf16),
                     preferred_element_type=f32).astype(bf16)

        hof = r.astype(f32)
        wof = c.astype(f32)
        for t0 in range(0, HW, tile):
            rT = lax.broadcasted_iota(jnp.int32, (H, tile), 0).astype(f32)
            acc = jnp.zeros((O, tile), f32)
            for k in range(n_taps):
                i, j = k // 3, k % 3
                py = hof[:, t0:t0 + tile] + float(i - 1) \
                    + offset[2 * k:2 * k + 1, t0:t0 + tile]
                px = wof[:, t0:t0 + tile] + float(j - 1) \
                    + offset[2 * k + 1:2 * k + 2, t0:t0 + tile]
                wy = jnp.maximum(1.0 - jnp.abs(rT - py), 0.0) \
                    * maskv[k:k + 1, t0:t0 + tile]
                wx = jnp.maximum(1.0 - jnp.abs(rT - px), 0.0)
                sk = (wy.reshape(H, 1, tile)
                      * wx.reshape(1, W, tile)).reshape(HW, tile)
                acc = acc + jnp.dot(xw[k * O:(k + 1) * O], sk.astype(bf16),
                                    preferred_element_type=f32)
            out_ref[0, :, t0:t0 + tile] = acc + b_dcn_ref[...]

    return _body


def kernel(x, weight, bias, conv_offset_mask_weight, conv_offset_mask_bias):
    B, cin, H, W = x.shape
    C = cin + 2
    O = weight.shape[0]
    HW = H * W
    n_taps = 9
    tile = _pick_tile(HW)

    h_i = jnp.broadcast_to(
        (jnp.arange(H, dtype=jnp.float32) / H)[:, None], (H, W))
    w_i = jnp.broadcast_to(
        (jnp.arange(W, dtype=jnp.float32) / W - 0.5)[None, :], (H, W))
    im_i = jnp.stack([h_i, w_i], axis=0)
    x_wi = jnp.concatenate(
        [x, jnp.broadcast_to(im_i[None], (B, 2, H, W))], axis=1)
    x_cm = x_wi.reshape(B, C, HW)

    w_all = jnp.transpose(conv_offset_mask_weight, (2, 3, 0, 1))
    w_all = w_all.reshape(n_taps, 3 * n_taps, C)
    w_all = jnp.pad(w_all, ((0, 0), (0, 32 - 3 * n_taps), (0, 0)))
    w_all = w_all.reshape(n_taps * 32, C)
    b_om_p = jnp.pad(conv_offset_mask_bias, (0, 32 - 3 * n_taps))
    b_om_p = b_om_p.reshape(32, 1)

    w_dcn_km = jnp.transpose(weight, (2, 3, 0, 1)).reshape(n_taps * O, C)
    w_dcn_km = w_dcn_km.astype(jnp.bfloat16)
    b_dcn_col = bias.reshape(O, 1)

    kh = jnp.repeat(jnp.arange(-1, 2), 3).astype(jnp.float32)
    kw = jnp.tile(jnp.arange(-1, 2), 3).astype(jnp.float32)
    k_i = jnp.stack([kh, kw], axis=0)
    base = (im_i[:, None, :, :]
            + k_i[:, :, None, None]).reshape(2 * n_taps, HW)

    body = _make_body(H, W, C, O, tile)
    out, idx, maskv = pl.pallas_call(
        body,
        grid=(B,),
        in_specs=[
            pl.BlockSpec((1, C, HW), lambda b: (b, 0, 0)),
            pl.BlockSpec((n_taps * 32, C), lambda b: (0, 0)),
            pl.BlockSpec((32, 1), lambda b: (0, 0)),
            pl.BlockSpec((n_taps * O, C), lambda b: (0, 0)),
            pl.BlockSpec((O, 1), lambda b: (0, 0)),
            pl.BlockSpec((2 * n_taps, HW), lambda b: (0, 0)),
        ],
        out_specs=[
            pl.BlockSpec((1, O, HW), lambda b: (b, 0, 0)),
            pl.BlockSpec((1, 2 * n_taps, HW), lambda b: (b, 0, 0)),
            pl.BlockSpec((1, n_taps, HW), lambda b: (b, 0, 0)),
        ],
        out_shape=[
            jax.ShapeDtypeStruct((B, O, HW), jnp.float32),
            jax.ShapeDtypeStruct((B, 2 * n_taps, HW), jnp.float32),
            jax.ShapeDtypeStruct((B, n_taps, HW), jnp.float32),
        ],
        compiler_params=pltpu.CompilerParams(
            dimension_semantics=("parallel",),
            vmem_limit_bytes=64 * 1024 * 1024),
    )(x_cm, w_all, b_om_p, w_dcn_km, b_dcn_col, base)

    return (out.reshape(B, O, H, W),
            idx.reshape(B, 2 * n_taps, H, W),
            maskv.reshape(B, n_taps, H, W))


# speedup vs baseline: 5.7101x; 1.1090x over previous
import jax
import jax.numpy as jnp
from jax import lax
from jax.experimental import pallas as pl
from jax.experimental.pallas import tpu as pltpu


def _pick_tile(hw):
    for cand in (512, 256, 128):
        if hw % cand == 0 and hw > cand:
            return cand
    return hw


def _make_body(H, W, C, O, tile):
    HW = H * W
    n_taps = 9

    def _body(x_ref, w_all_ref, b_om_ref, w_dcn_ref, b_dcn_ref, base_ref,
              out_ref, idx_ref, mask_ref):
        f32 = jnp.float32
        bf16 = jnp.bfloat16
        x = x_ref[0]

        y_all = jnp.dot(w_all_ref[...], x, preferred_element_type=f32)
        pos = lax.broadcasted_iota(jnp.int32, (1, HW), 1)
        r = pos // W
        c = pos - r * W
        om = jnp.broadcast_to(b_om_ref[...], (32, HW))
        for k in range(n_taps):
            dy, dx = k // 3 - 1, k % 3 - 1
            s = dy * W + dx
            yk = y_all[k * 32:(k + 1) * 32, :]
            rolled = pltpu.roll(yk, (-s) % HW, axis=1) if s else yk
            valid = ((r + dy >= 0) & (r + dy < H)
                     & (c + dx >= 0) & (c + dx < W))
            om = om + jnp.where(valid, rolled, 0.0)

        offset = om[:2 * n_taps]
        maskv = jax.nn.sigmoid(om[2 * n_taps:3 * n_taps])
        idx_ref[0] = base_ref[...] + offset
        mask_ref[0] = maskv

        xw = jnp.dot(w_dcn_ref[...], x.astype(bf16),
                     preferred_element_type=f32).astype(bf16)

        hof = r.astype(f32)
        wof = c.astype(f32)
        for t0 in range(0, HW, tile):
            rT = lax.broadcasted_iota(jnp.int32, (H, tile), 0).astype(f32)
            acc = jnp.zeros((O, tile), f32)
            for k in range(n_taps):
                i, j = k // 3, k % 3
                py = hof[:, t0:t0 + tile] + float(i - 1) \
                    + offset[2 * k:2 * k + 1, t0:t0 + tile]
                px = wof[:, t0:t0 + tile] + float(j - 1) \
                    + offset[2 * k + 1:2 * k + 2, t0:t0 + tile]
                wy = (jnp.maximum(1.0 - jnp.abs(rT - py), 0.0)
                      * maskv[k:k + 1, t0:t0 + tile]).astype(bf16)
                wx = jnp.maximum(1.0 - jnp.abs(rT - px), 0.0).astype(bf16)
                sk = (wy.reshape(H, 1, tile)
                      * wx.reshape(1, W, tile)).reshape(HW, tile)
                acc = acc + jnp.dot(xw[k * O:(k + 1) * O], sk,
                                    preferred_element_type=f32)
            out_ref[0, :, t0:t0 + tile] = acc + b_dcn_ref[...]

    return _body


def kernel(x, weight, bias, conv_offset_mask_weight, conv_offset_mask_bias):
    B, cin, H, W = x.shape
    C = cin + 2
    O = weight.shape[0]
    HW = H * W
    n_taps = 9
    tile = _pick_tile(HW)

    h_i = jnp.broadcast_to(
        (jnp.arange(H, dtype=jnp.float32) / H)[:, None], (H, W))
    w_i = jnp.broadcast_to(
        (jnp.arange(W, dtype=jnp.float32) / W - 0.5)[None, :], (H, W))
    im_i = jnp.stack([h_i, w_i], axis=0)
    x_wi = jnp.concatenate(
        [x, jnp.broadcast_to(im_i[None], (B, 2, H, W))], axis=1)
    x_cm = x_wi.reshape(B, C, HW)

    w_all = jnp.transpose(conv_offset_mask_weight, (2, 3, 0, 1))
    w_all = w_all.reshape(n_taps, 3 * n_taps, C)
    w_all = jnp.pad(w_all, ((0, 0), (0, 32 - 3 * n_taps), (0, 0)))
    w_all = w_all.reshape(n_taps * 32, C)
    b_om_p = jnp.pad(conv_offset_mask_bias, (0, 32 - 3 * n_taps))
    b_om_p = b_om_p.reshape(32, 1)

    w_dcn_km = jnp.transpose(weight, (2, 3, 0, 1)).reshape(n_taps * O, C)
    w_dcn_km = w_dcn_km.astype(jnp.bfloat16)
    b_dcn_col = bias.reshape(O, 1)

    kh = jnp.repeat(jnp.arange(-1, 2), 3).astype(jnp.float32)
    kw = jnp.tile(jnp.arange(-1, 2), 3).astype(jnp.float32)
    k_i = jnp.stack([kh, kw], axis=0)
    base = (im_i[:, None, :, :]
            + k_i[:, :, None, None]).reshape(2 * n_taps, HW)

    n_cores = 2
    bc = B // n_cores
    body = _make_body(H, W, C, O, tile)
    out, idx, maskv = pl.pallas_call(
        body,
        grid=(n_cores, bc),
        in_specs=[
            pl.BlockSpec((1, C, HW), lambda cc, i: (cc * bc + i, 0, 0)),
            pl.BlockSpec((n_taps * 32, C), lambda cc, i: (0, 0)),
            pl.BlockSpec((32, 1), lambda cc, i: (0, 0)),
            pl.BlockSpec((n_taps * O, C), lambda cc, i: (0, 0)),
            pl.BlockSpec((O, 1), lambda cc, i: (0, 0)),
            pl.BlockSpec((2 * n_taps, HW), lambda cc, i: (0, 0)),
        ],
        out_specs=[
            pl.BlockSpec((1, O, HW), lambda cc, i: (cc * bc + i, 0, 0)),
            pl.BlockSpec((1, 2 * n_taps, HW), lambda cc, i: (cc * bc + i, 0, 0)),
            pl.BlockSpec((1, n_taps, HW), lambda cc, i: (cc * bc + i, 0, 0)),
        ],
        out_shape=[
            jax.ShapeDtypeStruct((B, O, HW), jnp.float32),
            jax.ShapeDtypeStruct((B, 2 * n_taps, HW), jnp.float32),
            jax.ShapeDtypeStruct((B, n_taps, HW), jnp.float32),
        ],
        compiler_params=pltpu.CompilerParams(
            dimension_semantics=("parallel", "arbitrary"),
            vmem_limit_bytes=64 * 1024 * 1024),
    )(x_cm, w_all, b_om_p, w_dcn_km, b_dcn_col, base)

    return (out.reshape(B, O, H, W),
            idx.reshape(B, 2 * n_taps, H, W),
            maskv.reshape(B, n_taps, H, W))


# speedup vs baseline: 6.6973x; 1.1729x over previous
import jax
import jax.numpy as jnp
from jax import lax
from jax.experimental import pallas as pl
from jax.experimental.pallas import tpu as pltpu


def _pick_tile(hw):
    for cand in (512, 256, 128):
        if hw % cand == 0 and hw > cand:
            return cand
    return hw


def _make_body(H, W, cin, O, tile, nb):
    HW = H * W
    C = cin + 2
    n_taps = 9

    def _body(x_ref, imi_ref, w_all_ref, b_om_ref, w_dcn_ref, b_dcn_ref,
              base_ref, out_ref, idx_ref, mask_ref, xf_ref):
        f32 = jnp.float32
        bf16 = jnp.bfloat16
        pos = lax.broadcasted_iota(jnp.int32, (1, HW), 1)
        r = pos // W
        c = pos - r * W
        hof = r.astype(f32)
        wof = c.astype(f32)
        xf_ref[cin:C, :] = imi_ref[...]

        for ib in range(nb):
            xf_ref[0:cin, :] = x_ref[ib]
            x = xf_ref[...]

            y_all = jnp.dot(w_all_ref[...], x, preferred_element_type=f32)
            om = jnp.broadcast_to(b_om_ref[...], (32, HW))
            for k in range(n_taps):
                dy, dx = k // 3 - 1, k % 3 - 1
                s = dy * W + dx
                yk = y_all[k * 32:(k + 1) * 32, :]
                rolled = pltpu.roll(yk, (-s) % HW, axis=1) if s else yk
                valid = ((r + dy >= 0) & (r + dy < H)
                         & (c + dx >= 0) & (c + dx < W))
                om = om + jnp.where(valid, rolled, 0.0)

            offset = om[:2 * n_taps]
            maskv = jax.nn.sigmoid(om[2 * n_taps:3 * n_taps])
            idx_ref[ib] = base_ref[...] + offset
            mask_ref[ib] = maskv

            xw = jnp.dot(w_dcn_ref[...], x.astype(bf16),
                         preferred_element_type=f32).astype(bf16)

            for t0 in range(0, HW, tile):
                rT = lax.broadcasted_iota(jnp.int32, (H, tile), 0).astype(f32)
                acc = jnp.zeros((O, tile), f32)
                for k in range(n_taps):
                    i, j = k // 3, k % 3
                    py = hof[:, t0:t0 + tile] + float(i - 1) \
                        + offset[2 * k:2 * k + 1, t0:t0 + tile]
                    px = wof[:, t0:t0 + tile] + float(j - 1) \
                        + offset[2 * k + 1:2 * k + 2, t0:t0 + tile]
                    wy = (jnp.maximum(1.0 - jnp.abs(rT - py), 0.0)
                          * maskv[k:k + 1, t0:t0 + tile]).astype(bf16)
                    wx = jnp.maximum(1.0 - jnp.abs(rT - px), 0.0).astype(bf16)
                    sk = (wy.reshape(H, 1, tile)
                          * wx.reshape(1, W, tile)).reshape(HW, tile)
                    acc = acc + jnp.dot(xw[k * O:(k + 1) * O], sk,
                                        preferred_element_type=f32)
                out_ref[ib, :, t0:t0 + tile] = acc + b_dcn_ref[...]

    return _body


def kernel(x, weight, bias, conv_offset_mask_weight, conv_offset_mask_bias):
    B, cin, H, W = x.shape
    C = cin + 2
    O = weight.shape[0]
    HW = H * W
    n_taps = 9
    tile = _pick_tile(HW)
    nb = 4
    while B % nb:
        nb //= 2

    h_i = jnp.broadcast_to(
        (jnp.arange(H, dtype=jnp.float32) / H)[:, None], (H, W))
    w_i = jnp.broadcast_to(
        (jnp.arange(W, dtype=jnp.float32) / W - 0.5)[None, :], (H, W))
    im_i = jnp.stack([h_i, w_i], axis=0)
    imi_flat = im_i.reshape(2, HW)
    x_cm = x.reshape(B, cin, HW)

    w_all = jnp.transpose(conv_offset_mask_weight, (2, 3, 0, 1))
    w_all = w_all.reshape(n_taps, 3 * n_taps, C)
    w_all = jnp.pad(w_all, ((0, 0), (0, 32 - 3 * n_taps), (0, 0)))
    w_all = w_all.reshape(n_taps * 32, C)
    b_om_p = jnp.pad(conv_offset_mask_bias, (0, 32 - 3 * n_taps))
    b_om_p = b_om_p.reshape(32, 1)

    w_dcn_km = jnp.transpose(weight, (2, 3, 0, 1)).reshape(n_taps * O, C)
    w_dcn_km = w_dcn_km.astype(jnp.bfloat16)
    b_dcn_col = bias.reshape(O, 1)

    kh = jnp.repeat(jnp.arange(-1, 2), 3).astype(jnp.float32)
    kw = jnp.tile(jnp.arange(-1, 2), 3).astype(jnp.float32)
    k_i = jnp.stack([kh, kw], axis=0)
    base = (im_i[:, None, :, :]
            + k_i[:, :, None, None]).reshape(2 * n_taps, HW)

    body = _make_body(H, W, cin, O, tile, nb)
    out, idx, maskv = pl.pallas_call(
        body,
        grid=(B // nb,),
        in_specs=[
            pl.BlockSpec((nb, cin, HW), lambda b: (b, 0, 0)),
            pl.BlockSpec((2, HW), lambda b: (0, 0)),
            pl.BlockSpec((n_taps * 32, C), lambda b: (0, 0)),
            pl.BlockSpec((32, 1), lambda b: (0, 0)),
            pl.BlockSpec((n_taps * O, C), lambda b: (0, 0)),
            pl.BlockSpec((O, 1), lambda b: (0, 0)),
            pl.BlockSpec((2 * n_taps, HW), lambda b: (0, 0)),
        ],
        out_specs=[
            pl.BlockSpec((nb, O, HW), lambda b: (b, 0, 0)),
            pl.BlockSpec((nb, 2 * n_taps, HW), lambda b: (b, 0, 0)),
            pl.BlockSpec((nb, n_taps, HW), lambda b: (b, 0, 0)),
        ],
        out_shape=[
            jax.ShapeDtypeStruct((B, O, HW), jnp.float32),
            jax.ShapeDtypeStruct((B, 2 * n_taps, HW), jnp.float32),
            jax.ShapeDtypeStruct((B, n_taps, HW), jnp.float32),
        ],
        scratch_shapes=[pltpu.VMEM((C, HW), jnp.float32)],
        compiler_params=pltpu.CompilerParams(
            dimension_semantics=("arbitrary",),
            vmem_limit_bytes=64 * 1024 * 1024),
    )(x_cm, imi_flat, w_all, b_om_p, w_dcn_km, b_dcn_col, base)

    return (out.reshape(B, O, H, W),
            idx.reshape(B, 2 * n_taps, H, W),
            maskv.reshape(B, n_taps, H, W))
